# baseline (speedup 1.0000x reference)
import functools

import jax
import jax.numpy as jnp
from jax import lax
from jax.experimental import pallas as pl
from jax.experimental.pallas import tpu as pltpu
from jax.experimental.pallas import tpu_sc as plsc

_B, _C, _N, _K, _OUT = 16, 64, 2048, 16, 64
_EPS = 1e-5
_PTS = _B * _N
_BNK = _PTS * _K
_NEG = -3.0e38

_RT = 1024


def _topk_body(boffs, xfull_ref, xtile_ref, xx_ref, idx_ref):
    b = pl.program_id(0) + boffs
    nt = pl.program_id(1)
    xb = xfull_ref[0]
    xt = xtile_ref[0]
    xx = xx_ref[0]
    xxt = jnp.sum(xt * xt, axis=0, keepdims=True)
    ip = lax.dot_general(xt, xb, (((0,), (0,)), ((), ())),
                         preferred_element_type=jnp.float32)
    vals = 2.0 * ip - xxt.T - xx
    base = b * _N
    ng = _N // 128
    vs = [vals[:, g * 128:(g + 1) * 128] for g in range(ng)]
    lane = lax.broadcasted_iota(jnp.int32, (_RT, 128), 1).astype(jnp.float32)
    row = lax.broadcasted_iota(jnp.int32, (_RT, 1), 0).astype(jnp.float32)
    big = jnp.float32(3.0e38)
    neg = jnp.float32(_NEG)

    a0 = row + (nt * _RT)
    cols = [a0]
    vs = [jnp.where(lane == a0 - (g * 128.0), neg, vs[g]) for g in range(ng)]

    depth = 5
    mv = [jnp.full((_RT, 128), _NEG, jnp.float32) for _ in range(depth)]
    ag = [jnp.zeros((_RT, 128), jnp.float32) for _ in range(depth)]
    for g in range(ng):
        v = vs[g]
        cio = lane + (g * 128.0)
        for i in range(depth):
            c = v > mv[i]
            nv, na = jnp.where(c, v, mv[i]), jnp.where(c, cio, ag[i])
            v, cio = jnp.where(c, mv[i], v), jnp.where(c, ag[i], cio)
            mv[i], ag[i] = nv, na
    for t in range(1, _K):
        m = jnp.max(mv[0], axis=1, keepdims=True)
        cand = jnp.where(mv[0] == m, ag[0], big)
        a = jnp.min(cand, axis=1, keepdims=True)
        cols.append(a)
        lf = a - 128.0 * jnp.trunc(a * (1.0 / 128.0))
        c = lane == lf
        for i in range(depth - 1):
            mv[i] = jnp.where(c, mv[i + 1], mv[i])
            ag[i] = jnp.where(c, ag[i + 1], ag[i])
        mv[depth - 1] = jnp.where(c, neg, mv[depth - 1])

    idx_ref[0, :, :] = jnp.concatenate(cols, axis=1).astype(jnp.int32) + base

    bad = jnp.min(mv[0]) < jnp.float32(-1.0e37)

    @pl.when(bad)
    def _full_extraction():
        ws = [jnp.where(lane == a0 - (g * 128.0),
                        neg, vals[:, g * 128:(g + 1) * 128])
              for g in range(ng)]
        fcols = [a0]
        for t in range(1, _K):
            fmv = ws[0]
            fag = jnp.zeros((_RT, 128), jnp.float32)
            for g in range(1, ng):
                cg = ws[g] > fmv
                fmv = jnp.where(cg, ws[g], fmv)
                fag = jnp.where(cg, jnp.float32(g * 128.0), fag)
            fm = jnp.max(fmv, axis=1, keepdims=True)
            fcand = jnp.where(fmv == fm, fag + lane, big)
            fa = jnp.min(fcand, axis=1, keepdims=True)
            fcols.append(fa)
            ws = [jnp.where(lane == fa - (g * 128.0), neg, ws[g])
                  for g in range(ng)]
        idx_ref[0, :, :] = (jnp.concatenate(fcols, axis=1).astype(jnp.int32)
                            + base)


def _topk(x, xx, boffs):
    nb = x.shape[0]
    return pl.pallas_call(
        functools.partial(_topk_body, boffs),
        grid=(nb, _N // _RT),
        in_specs=[
            pl.BlockSpec((1, _C, _N), lambda b, t: (b, 0, 0)),
            pl.BlockSpec((1, _C, _RT), lambda b, t: (b, 0, t)),
            pl.BlockSpec((1, 1, _N), lambda b, t: (b, 0, 0)),
        ],
        out_specs=pl.BlockSpec((1, _RT, _K), lambda b, t: (b, t, 0)),
        out_shape=jax.ShapeDtypeStruct((nb, _N, _K), jnp.int32),
    )(x, x, xx)


def _proj_body(x_ref, w_ref, y1_ref, dd_ref, xx_ref):
    xb = x_ref[0]
    w1 = w_ref[:, :_C]
    wd = w_ref[:, _C:] - w1
    dn = (((0,), (1,)), ((), ()))
    y1_ref[0, :, :_OUT] = lax.dot_general(xb, w1, dn,
                                          preferred_element_type=jnp.float32)
    y1_ref[0, :, _OUT:] = jnp.zeros((_N, 128 - _OUT), jnp.float32)
    dd_ref[0] = lax.dot_general(xb, wd, dn, preferred_element_type=jnp.float32)
    xx_ref[0] = jnp.sum(xb * xb, axis=0, keepdims=True)


def _proj(x, W):
    return pl.pallas_call(
        _proj_body,
        grid=(_B,),
        in_specs=[
            pl.BlockSpec((1, _C, _N), lambda b: (b, 0, 0)),
            pl.BlockSpec((_OUT, 2 * _C), lambda b: (0, 0)),
        ],
        out_specs=[
            pl.BlockSpec((1, _N, 128), lambda b: (b, 0, 0)),
            pl.BlockSpec((1, _N, _OUT), lambda b: (b, 0, 0)),
            pl.BlockSpec((1, 1, _N), lambda b: (b, 0, 0)),
        ],
        out_shape=[
            jax.ShapeDtypeStruct((_B, _N, 128), jnp.float32),
            jax.ShapeDtypeStruct((_B, _N, _OUT), jnp.float32),
            jax.ShapeDtypeStruct((_B, 1, _N), jnp.float32),
        ],
    )(x, W)


_INFO = plsc.get_sparse_core_info()
_NC, _NS, _L = _INFO.num_cores, _INFO.num_subcores, _INFO.num_lanes
_NW = _NC * _NS
_PPW = _PTS // _NW
_PCH = 8
_NITER = _PPW // _PCH


def _make_sc(npts):
    ppw = npts // _NW

    def _sc_body(y1_hbm, idx_hbm, out_hbm, idx_v, rows_v, red_v, sem):
        wid = lax.axis_index("s") * _NC + lax.axis_index("c")
        base = wid * ppw

        def chunk(ci, carry):
            row0 = base + ci * _PCH
            pltpu.sync_copy(idx_hbm.at[pl.ds(row0 * _K, _PCH * _K)], idx_v)
            pltpu.async_copy(y1_hbm.at[idx_v], rows_v, sem).wait()

            def point(p, c2):
                r0 = p * _K
                for g in range(_OUT // _L):
                    col = pl.ds(g * _L, _L)
                    v = rows_v[r0, col]
                    mx = v
                    mn = v
                    s1 = v
                    s2 = v * v
                    for k in range(1, _K):
                        v = rows_v[r0 + k, col]
                        mx = jnp.maximum(mx, v)
                        mn = jnp.minimum(mn, v)
                        s1 = s1 + v
                        s2 = s2 + v * v
                    red_v[p, pl.ds(0 * _OUT + g * _L, _L)] = mx
                    red_v[p, pl.ds(1 * _OUT + g * _L, _L)] = mn
                    red_v[p, pl.ds(2 * _OUT + g * _L, _L)] = s1
                    red_v[p, pl.ds(3 * _OUT + g * _L, _L)] = s2
                return c2

            lax.fori_loop(0, _PCH, point, 0)
            pltpu.sync_copy(red_v, out_hbm.at[pl.ds(row0, _PCH)])
            return carry

        lax.fori_loop(0, ppw // _PCH, chunk, 0)

    return functools.partial(
        pl.kernel,
        out_type=jax.ShapeDtypeStruct((npts, 4 * _OUT), jnp.float32),
        scratch_types=[
            pltpu.VMEM((_PCH * _K,), jnp.int32),
            pltpu.VMEM((_PCH * _K, 128), jnp.float32),
            pltpu.VMEM((_PCH, 4 * _OUT), jnp.float32),
            pltpu.SemaphoreType.DMA,
        ],
        mesh=plsc.VectorSubcoreMesh(core_axis_name="c", subcore_axis_name="s"),
    )(_sc_body)


_sc_half = _make_sc(_PTS // 2)


_SCH = 2048


def _stats_body(red_ref, dd_ref, st_ref):
    c = pl.program_id(0)

    @pl.when(c == 0)
    def _init():
        st_ref[...] = jnp.zeros((2, _OUT), jnp.float32)

    s1 = red_ref[:, 2, :]
    s2 = red_ref[:, 3, :]
    dd = dd_ref[...]
    kf = jnp.float32(_K)
    hs = jnp.sum(s1 + kf * dd, axis=0, keepdims=True)
    hq = jnp.sum(s2 + 2.0 * dd * s1 + kf * dd * dd, axis=0, keepdims=True)
    st_ref[0:1, :] += hs
    st_ref[1:2, :] += hq

    @pl.when(c == _PTS // _SCH - 1)
    def _finish():
        inv = jnp.float32(1.0 / _BNK)
        mean = st_ref[0:1, :] * inv
        var = st_ref[1:2, :] * inv - mean * mean
        st_ref[0:1, :] = mean
        st_ref[1:2, :] = lax.rsqrt(var + _EPS)


def _stats(red, ddf):
    return pl.pallas_call(
        _stats_body,
        grid=(_PTS // _SCH,),
        in_specs=[
            pl.BlockSpec((_SCH, 4, _OUT), lambda c: (c, 0, 0)),
            pl.BlockSpec((_SCH, _OUT), lambda c: (c, 0)),
        ],
        out_specs=pl.BlockSpec((2, _OUT), lambda c: (0, 0)),
        out_shape=jax.ShapeDtypeStruct((2, _OUT), jnp.float32),
    )(red, ddf)


_TN = 512


def _final_body(red_ref, dd_ref, st_ref, gb_ref, out_ref):
    mp = red_ref[0, :, 0, :]
    mn = red_ref[0, :, 1, :]
    dd = dd_ref[0]
    gamma = gb_ref[0:1, :]
    beta = gb_ref[1:2, :]
    mean = st_ref[0:1, :]
    rstd = st_ref[1:2, :]
    sel = jnp.where(gamma >= 0.0, mp, mn) + dd
    y = (sel - mean) * (rstd * gamma) + beta
    y = jnp.where(y >= 0.0, y, 0.2 * y)
    out_ref[0] = y.T


def _final(red4, dd, stats, gb):
    return pl.pallas_call(
        _final_body,
        grid=(_B, _N // _TN),
        in_specs=[
            pl.BlockSpec((1, _TN, 4, _OUT), lambda b, t: (b, t, 0, 0)),
            pl.BlockSpec((1, _TN, _OUT), lambda b, t: (b, t, 0)),
            pl.BlockSpec((2, _OUT), lambda b, t: (0, 0)),
            pl.BlockSpec((2, _OUT), lambda b, t: (0, 0)),
        ],
        out_specs=pl.BlockSpec((1, _OUT, _TN), lambda b, t: (b, 0, t)),
        out_shape=jax.ShapeDtypeStruct((_B, _OUT, _N), jnp.float32),
    )(red4, dd, stats, gb)


def kernel(x, W, gamma, beta):
    h = _B // 2
    y1, dd, xx = _proj(x, W)
    y1f = y1.reshape(_PTS, 128)
    idx1 = _topk(x[:h], xx[:h], 0)
    red1 = _sc_half(y1f, idx1.reshape(_PTS * _K // 2))
    idx2 = _topk(x[h:], xx[h:], h)
    red2 = _sc_half(y1f, idx2.reshape(_PTS * _K // 2))
    red = jnp.concatenate([red1, red2], axis=0).reshape(_PTS, 4, _OUT)
    stats = _stats(red, dd.reshape(_PTS, _OUT))
    gb = jnp.stack([gamma, beta])
    return _final(red.reshape(_B, _N, 4, _OUT), dd, stats, gb)


# speedup vs baseline: 1.1316x; 1.1316x over previous
import functools

import jax
import jax.numpy as jnp
from jax import lax
from jax.experimental import pallas as pl
from jax.experimental.pallas import tpu as pltpu
from jax.experimental.pallas import tpu_sc as plsc

_B, _C, _N, _K, _OUT = 16, 64, 2048, 16, 64
_EPS = 1e-5
_PTS = _B * _N
_BNK = _PTS * _K
_NEG = -3.0e38

_RT = 512


def _topk_body(boffs, xfull_ref, xtile_ref, xx_ref, idx_ref):
    b = pl.program_id(0) + boffs
    nt = pl.program_id(1)
    xb = xfull_ref[0]
    xt = xtile_ref[0]
    xx = xx_ref[0]
    xxt = jnp.sum(xt * xt, axis=0, keepdims=True)
    ip = lax.dot_general(xt, xb, (((0,), (0,)), ((), ())),
                         preferred_element_type=jnp.float32)
    vals = 2.0 * ip - xxt.T - xx
    base = b * _N
    ng = _N // 128
    vs = [vals[:, g * 128:(g + 1) * 128] for g in range(ng)]
    lane = lax.broadcasted_iota(jnp.int32, (_RT, 128), 1).astype(jnp.float32)
    row = lax.broadcasted_iota(jnp.int32, (_RT, 1), 0).astype(jnp.float32)
    big = jnp.float32(3.0e38)
    neg = jnp.float32(_NEG)

    a0 = row + (nt * _RT)
    cols = [a0]
    vs = [jnp.where(lane == a0 - (g * 128.0), neg, vs[g]) for g in range(ng)]

    depth = 5
    mv = [jnp.full((_RT, 128), _NEG, jnp.float32) for _ in range(depth)]
    ag = [jnp.zeros((_RT, 128), jnp.float32) for _ in range(depth)]
    for g in range(ng):
        v = vs[g]
        cio = lane + (g * 128.0)
        for i in range(depth):
            c = v > mv[i]
            nv, na = jnp.where(c, v, mv[i]), jnp.where(c, cio, ag[i])
            v, cio = jnp.where(c, mv[i], v), jnp.where(c, ag[i], cio)
            mv[i], ag[i] = nv, na
    for t in range(1, _K):
        m = jnp.max(mv[0], axis=1, keepdims=True)
        cand = jnp.where(mv[0] == m, ag[0], big)
        a = jnp.min(cand, axis=1, keepdims=True)
        cols.append(a)
        lf = a - 128.0 * jnp.trunc(a * (1.0 / 128.0))
        c = lane == lf
        for i in range(depth - 1):
            mv[i] = jnp.where(c, mv[i + 1], mv[i])
            ag[i] = jnp.where(c, ag[i + 1], ag[i])
        mv[depth - 1] = jnp.where(c, neg, mv[depth - 1])

    idx_ref[0, :, :] = jnp.concatenate(cols, axis=1).astype(jnp.int32) + base

    bad = jnp.min(mv[0]) < jnp.float32(-1.0e37)

    @pl.when(bad)
    def _full_extraction():
        ws = [jnp.where(lane == a0 - (g * 128.0),
                        neg, vals[:, g * 128:(g + 1) * 128])
              for g in range(ng)]
        fcols = [a0]
        for t in range(1, _K):
            fmv = ws[0]
            fag = jnp.zeros((_RT, 128), jnp.float32)
            for g in range(1, ng):
                cg = ws[g] > fmv
                fmv = jnp.where(cg, ws[g], fmv)
                fag = jnp.where(cg, jnp.float32(g * 128.0), fag)
            fm = jnp.max(fmv, axis=1, keepdims=True)
            fcand = jnp.where(fmv == fm, fag + lane, big)
            fa = jnp.min(fcand, axis=1, keepdims=True)
            fcols.append(fa)
            ws = [jnp.where(lane == fa - (g * 128.0), neg, ws[g])
                  for g in range(ng)]
        idx_ref[0, :, :] = (jnp.concatenate(fcols, axis=1).astype(jnp.int32)
                            + base)


def _topk(x, xx, boffs):
    nb = x.shape[0]
    return pl.pallas_call(
        functools.partial(_topk_body, boffs),
        grid=(nb, _N // _RT),
        in_specs=[
            pl.BlockSpec((1, _C, _N), lambda b, t: (b, 0, 0)),
            pl.BlockSpec((1, _C, _RT), lambda b, t: (b, 0, t)),
            pl.BlockSpec((1, 1, _N), lambda b, t: (b, 0, 0)),
        ],
        out_specs=pl.BlockSpec((1, _RT, _K), lambda b, t: (b, t, 0)),
        out_shape=jax.ShapeDtypeStruct((nb, _N, _K), jnp.int32),
    )(x, x, xx)


def _proj_body(x_ref, w_ref, y1_ref, dd_ref, xx_ref):
    xb = x_ref[0]
    w1 = w_ref[:, :_C]
    wd = w_ref[:, _C:] - w1
    dn = (((0,), (1,)), ((), ()))
    y1_ref[0, :, :_OUT] = lax.dot_general(xb, w1, dn,
                                          preferred_element_type=jnp.float32)
    y1_ref[0, :, _OUT:] = jnp.zeros((_N, 128 - _OUT), jnp.float32)
    dd_ref[0] = lax.dot_general(xb, wd, dn, preferred_element_type=jnp.float32)
    xx_ref[0] = jnp.sum(xb * xb, axis=0, keepdims=True)


def _proj(x, W):
    return pl.pallas_call(
        _proj_body,
        grid=(_B,),
        in_specs=[
            pl.BlockSpec((1, _C, _N), lambda b: (b, 0, 0)),
            pl.BlockSpec((_OUT, 2 * _C), lambda b: (0, 0)),
        ],
        out_specs=[
            pl.BlockSpec((1, _N, 128), lambda b: (b, 0, 0)),
            pl.BlockSpec((1, _N, _OUT), lambda b: (b, 0, 0)),
            pl.BlockSpec((1, 1, _N), lambda b: (b, 0, 0)),
        ],
        out_shape=[
            jax.ShapeDtypeStruct((_B, _N, 128), jnp.float32),
            jax.ShapeDtypeStruct((_B, _N, _OUT), jnp.float32),
            jax.ShapeDtypeStruct((_B, 1, _N), jnp.float32),
        ],
    )(x, W)


_INFO = plsc.get_sparse_core_info()
_NC, _NS, _L = _INFO.num_cores, _INFO.num_subcores, _INFO.num_lanes
_NW = _NC * _NS
_PPW = _PTS // _NW
_PCH = 8
_NITER = _PPW // _PCH


def _make_sc(npts):
    ppw = npts // _NW
    niter = ppw // _PCH
    assert niter % 2 == 0 and niter >= 4

    def _sc_body(y1_hbm, idx_hbm, out_hbm,
                 idx_v0, idx_v1, rows_v0, rows_v1, red_v, sem0, sem1):
        wid = lax.axis_index("s") * _NC + lax.axis_index("c")
        base = wid * ppw
        idxv = (idx_v0, idx_v1)
        rowsv = (rows_v0, rows_v1)
        sems = (sem0, sem1)

        def fire(ci, par):
            row0 = base + ci * _PCH
            pltpu.sync_copy(idx_hbm.at[pl.ds(row0 * _K, _PCH * _K)],
                            idxv[par])
            pltpu.async_copy(y1_hbm.at[idxv[par]], rowsv[par], sems[par])

        fire(0, 0)
        fire(1, 1)

        def pair(i, carry):
            ci0 = i * 2
            for par in range(2):
                ci = ci0 + par
                row0 = base + ci * _PCH
                pltpu.make_async_copy(y1_hbm.at[idxv[par]], rowsv[par],
                                      sems[par]).wait()
                rv = rowsv[par]

                def point(p, c2, rv=rv):
                    r0 = p * _K
                    for g in range(_OUT // _L):
                        col = pl.ds(g * _L, _L)
                        v = rv[r0, col]
                        mx = v
                        mn = v
                        s1 = v
                        s2 = v * v
                        for k in range(1, _K):
                            v = rv[r0 + k, col]
                            mx = jnp.maximum(mx, v)
                            mn = jnp.minimum(mn, v)
                            s1 = s1 + v
                            s2 = s2 + v * v
                        red_v[p, pl.ds(0 * _OUT + g * _L, _L)] = mx
                        red_v[p, pl.ds(1 * _OUT + g * _L, _L)] = mn
                        red_v[p, pl.ds(2 * _OUT + g * _L, _L)] = s1
                        red_v[p, pl.ds(3 * _OUT + g * _L, _L)] = s2
                    return c2

                lax.fori_loop(0, _PCH, point, 0)
                pltpu.sync_copy(red_v, out_hbm.at[pl.ds(row0, _PCH)])

                @pl.when(ci + 2 < niter)
                def _prefetch(ci=ci, par=par):
                    fire(ci + 2, par)
            return carry

        lax.fori_loop(0, niter // 2, pair, 0)

    return functools.partial(
        pl.kernel,
        out_type=jax.ShapeDtypeStruct((npts, 4 * _OUT), jnp.float32),
        scratch_types=[
            pltpu.VMEM((_PCH * _K,), jnp.int32),
            pltpu.VMEM((_PCH * _K,), jnp.int32),
            pltpu.VMEM((_PCH * _K, 128), jnp.float32),
            pltpu.VMEM((_PCH * _K, 128), jnp.float32),
            pltpu.VMEM((_PCH, 4 * _OUT), jnp.float32),
            pltpu.SemaphoreType.DMA,
            pltpu.SemaphoreType.DMA,
        ],
        mesh=plsc.VectorSubcoreMesh(core_axis_name="c", subcore_axis_name="s"),
    )(_sc_body)


_sc_half = _make_sc(_PTS // 2)


_SCH = 2048


def _stats_body(red_ref, dd_ref, st_ref):
    c = pl.program_id(0)

    @pl.when(c == 0)
    def _init():
        st_ref[...] = jnp.zeros((2, _OUT), jnp.float32)

    s1 = red_ref[:, 2, :]
    s2 = red_ref[:, 3, :]
    dd = dd_ref[...]
    kf = jnp.float32(_K)
    hs = jnp.sum(s1 + kf * dd, axis=0, keepdims=True)
    hq = jnp.sum(s2 + 2.0 * dd * s1 + kf * dd * dd, axis=0, keepdims=True)
    st_ref[0:1, :] += hs
    st_ref[1:2, :] += hq

    @pl.when(c == _PTS // _SCH - 1)
    def _finish():
        inv = jnp.float32(1.0 / _BNK)
        mean = st_ref[0:1, :] * inv
        var = st_ref[1:2, :] * inv - mean * mean
        st_ref[0:1, :] = mean
        st_ref[1:2, :] = lax.rsqrt(var + _EPS)


def _stats(red, ddf):
    return pl.pallas_call(
        _stats_body,
        grid=(_PTS // _SCH,),
        in_specs=[
            pl.BlockSpec((_SCH, 4, _OUT), lambda c: (c, 0, 0)),
            pl.BlockSpec((_SCH, _OUT), lambda c: (c, 0)),
        ],
        out_specs=pl.BlockSpec((2, _OUT), lambda c: (0, 0)),
        out_shape=jax.ShapeDtypeStruct((2, _OUT), jnp.float32),
    )(red, ddf)


_TN = 512


def _final_body(red_ref, dd_ref, st_ref, gb_ref, out_ref):
    mp = red_ref[0, :, 0, :]
    mn = red_ref[0, :, 1, :]
    dd = dd_ref[0]
    gamma = gb_ref[0:1, :]
    beta = gb_ref[1:2, :]
    mean = st_ref[0:1, :]
    rstd = st_ref[1:2, :]
    sel = jnp.where(gamma >= 0.0, mp, mn) + dd
    y = (sel - mean) * (rstd * gamma) + beta
    y = jnp.where(y >= 0.0, y, 0.2 * y)
    out_ref[0] = y.T


def _final(red4, dd, stats, gb):
    return pl.pallas_call(
        _final_body,
        grid=(_B, _N // _TN),
        in_specs=[
            pl.BlockSpec((1, _TN, 4, _OUT), lambda b, t: (b, t, 0, 0)),
            pl.BlockSpec((1, _TN, _OUT), lambda b, t: (b, t, 0)),
            pl.BlockSpec((2, _OUT), lambda b, t: (0, 0)),
            pl.BlockSpec((2, _OUT), lambda b, t: (0, 0)),
        ],
        out_specs=pl.BlockSpec((1, _OUT, _TN), lambda b, t: (b, 0, t)),
        out_shape=jax.ShapeDtypeStruct((_B, _OUT, _N), jnp.float32),
    )(red4, dd, stats, gb)


def kernel(x, W, gamma, beta):
    h = _B // 2
    y1, dd, xx = _proj(x, W)
    y1f = y1.reshape(_PTS, 128)
    idx1 = _topk(x[:h], xx[:h], 0)
    red1 = _sc_half(y1f, idx1.reshape(_PTS * _K // 2))
    idx2 = _topk(x[h:], xx[h:], h)
    red2 = _sc_half(y1f, idx2.reshape(_PTS * _K // 2))
    red = jnp.concatenate([red1, red2], axis=0).reshape(_PTS, 4, _OUT)
    stats = _stats(red, dd.reshape(_PTS, _OUT))
    gb = jnp.stack([gamma, beta])
    return _final(red.reshape(_B, _N, 4, _OUT), dd, stats, gb)
